# initial kernel scaffold (unmeasured)
import jax
import jax.numpy as jnp
from jax import lax
from jax.experimental import pallas as pl
from jax.experimental.pallas import tpu as pltpu

P = 32
R_HOPS = P // 2
L_HOPS = P // 2 - 1
BF_STAGES = 5


def kernel(x, w_mat):
    m_per, k = x.shape
    _, n_per = w_mat.shape
    m_total = P * m_per

    def body(x_ref, w_ref, out_ref, buf_r, buf_l,
             send_r, recv_r, send_l, recv_l,
             bf_src, bf_dst, bf_send, bf_recv):
        my = lax.axis_index("i")
        right = lax.rem(my + 1, P)
        left = lax.rem(my + P - 1, P)

        barrier = pltpu.get_barrier_semaphore()
        for nbr in (left, right):
            pl.semaphore_signal(barrier, inc=1, device_id=(nbr,),
                                device_id_type=pl.DeviceIdType.MESH)
        pl.semaphore_wait(barrier, 2)

        w = w_ref[:, :]

        def gemm_relu(chunk):
            y = jnp.dot(chunk, w, preferred_element_type=jnp.float32)
            return jnp.maximum(y, 0.0)

        y0 = gemm_relu(x_ref[:, :])
        out_ref[pl.ds(my * m_per, m_per), :] = y0
        amax = jnp.max(y0)

        for h in range(R_HOPS):
            src_r = x_ref if h == 0 else buf_r.at[(h - 1) % 2]
            rdma_r = pltpu.make_async_remote_copy(
                src_ref=src_r,
                dst_ref=buf_r.at[h % 2],
                send_sem=send_r.at[h],
                recv_sem=recv_r.at[h],
                device_id=(right,),
                device_id_type=pl.DeviceIdType.MESH,
            )
            rdma_r.start()
            if h < L_HOPS:
                src_l = x_ref if h == 0 else buf_l.at[(h - 1) % 2]
                rdma_l = pltpu.make_async_remote_copy(
                    src_ref=src_l,
                    dst_ref=buf_l.at[h % 2],
                    send_sem=send_l.at[h],
                    recv_sem=recv_l.at[h],
                    device_id=(left,),
                    device_id_type=pl.DeviceIdType.MESH,
                )
                rdma_l.start()

            rdma_r.wait()
            origin_r = lax.rem(my - 1 - h + P, P)
            y_r = gemm_relu(buf_r[h % 2])
            out_ref[pl.ds(origin_r * m_per, m_per), :] = y_r
            amax = jnp.maximum(amax, jnp.max(y_r))

            if h < L_HOPS:
                rdma_l.wait()
                origin_l = lax.rem(my + 1 + h, P)
                y_l = gemm_relu(buf_l[h % 2])
                out_ref[pl.ds(origin_l * m_per, m_per), :] = y_l
                amax = jnp.maximum(amax, jnp.max(y_l))

        for s in range(BF_STAGES):
            partner = my ^ (1 << s)
            bf_src[s] = jnp.full((8, 128), amax, dtype=jnp.float32)
            exch = pltpu.make_async_remote_copy(
                src_ref=bf_src.at[s],
                dst_ref=bf_dst.at[s],
                send_sem=bf_send.at[s],
                recv_sem=bf_recv.at[s],
                device_id=(partner,),
                device_id_type=pl.DeviceIdType.MESH,
            )
            exch.start()
            exch.wait()
            amax = jnp.maximum(amax, bf_dst[s, 0, 0])

        scale = amax / 448.0
        a = out_ref[:, :] / scale
        q = a.astype(jnp.float8_e4m3fn).astype(jnp.float32)
        out_ref[:, :] = q * scale

    return pl.pallas_call(
        body,
        out_shape=jax.ShapeDtypeStruct((m_total, n_per), jnp.float32),
        in_specs=[
            pl.BlockSpec(memory_space=pltpu.VMEM),
            pl.BlockSpec(memory_space=pltpu.VMEM),
        ],
        out_specs=pl.BlockSpec(memory_space=pltpu.VMEM),
        scratch_shapes=[
            pltpu.VMEM((2, m_per, k), jnp.float32),
            pltpu.VMEM((2, m_per, k), jnp.float32),
            pltpu.SemaphoreType.DMA((R_HOPS,)),
            pltpu.SemaphoreType.DMA((R_HOPS,)),
            pltpu.SemaphoreType.DMA((L_HOPS,)),
            pltpu.SemaphoreType.DMA((L_HOPS,)),
            pltpu.VMEM((BF_STAGES, 8, 128), jnp.float32),
            pltpu.VMEM((BF_STAGES, 8, 128), jnp.float32),
            pltpu.SemaphoreType.DMA((BF_STAGES,)),
            pltpu.SemaphoreType.DMA((BF_STAGES,)),
        ],
        compiler_params=pltpu.CompilerParams(collective_id=0),
    )(x, w_mat)


# baseline (device time: 798175 ns/iter reference)
import jax
import jax.numpy as jnp
from jax import lax
from jax.experimental import pallas as pl
from jax.experimental.pallas import tpu as pltpu

P = 32
R_HOPS = P // 2
L_HOPS = P // 2 - 1
BF_STAGES = 5


def kernel(x, w_mat):
    m_per, k = x.shape
    _, n_per = w_mat.shape
    m_total = P * m_per

    def body(x_ref, w_ref, out_ref, buf_r, buf_l,
             send_r, recv_r, send_l, recv_l,
             credit_r, credit_l,
             bf_src, bf_dst, bf_send, bf_recv):
        my = lax.axis_index("i")
        right = lax.rem(my + 1, P)
        left = lax.rem(my + P - 1, P)

        barrier = pltpu.get_barrier_semaphore()
        for nbr in (left, right):
            pl.semaphore_signal(barrier, inc=1, device_id=(nbr,),
                                device_id_type=pl.DeviceIdType.MESH)
        pl.semaphore_wait(barrier, 2)

        w = w_ref[:, :]

        def gemm_relu(chunk):
            y = jnp.dot(chunk, w, preferred_element_type=jnp.float32)
            return jnp.maximum(y, 0.0)

        y0 = gemm_relu(x_ref[:, :])
        out_ref[pl.ds(my * m_per, m_per), :] = y0
        amax = jnp.max(y0)

        for h in range(R_HOPS):
            if h >= 2:
                pl.semaphore_wait(credit_r, 1)
            if 2 <= h < L_HOPS:
                pl.semaphore_wait(credit_l, 1)

            src_r = x_ref if h == 0 else buf_r.at[(h - 1) % 2]
            rdma_r = pltpu.make_async_remote_copy(
                src_ref=src_r,
                dst_ref=buf_r.at[h % 2],
                send_sem=send_r.at[h],
                recv_sem=recv_r.at[h],
                device_id=(right,),
                device_id_type=pl.DeviceIdType.MESH,
            )
            rdma_r.start()
            if h < L_HOPS:
                src_l = x_ref if h == 0 else buf_l.at[(h - 1) % 2]
                rdma_l = pltpu.make_async_remote_copy(
                    src_ref=src_l,
                    dst_ref=buf_l.at[h % 2],
                    send_sem=send_l.at[h],
                    recv_sem=recv_l.at[h],
                    device_id=(left,),
                    device_id_type=pl.DeviceIdType.MESH,
                )
                rdma_l.start()

            rdma_r.wait()
            if 1 <= h <= R_HOPS - 2:
                pl.semaphore_signal(credit_r, inc=1, device_id=(left,),
                                    device_id_type=pl.DeviceIdType.MESH)
            origin_r = lax.rem(my - 1 - h + P, P)
            y_r = gemm_relu(buf_r[h % 2])
            out_ref[pl.ds(origin_r * m_per, m_per), :] = y_r
            amax = jnp.maximum(amax, jnp.max(y_r))

            if h < L_HOPS:
                rdma_l.wait()
                if 1 <= h <= L_HOPS - 2:
                    pl.semaphore_signal(credit_l, inc=1, device_id=(right,),
                                        device_id_type=pl.DeviceIdType.MESH)
                origin_l = lax.rem(my + 1 + h, P)
                y_l = gemm_relu(buf_l[h % 2])
                out_ref[pl.ds(origin_l * m_per, m_per), :] = y_l
                amax = jnp.maximum(amax, jnp.max(y_l))

        for s in range(BF_STAGES):
            partner = my ^ (1 << s)
            bf_src[s] = jnp.full((8, 128), amax, dtype=jnp.float32)
            exch = pltpu.make_async_remote_copy(
                src_ref=bf_src.at[s],
                dst_ref=bf_dst.at[s],
                send_sem=bf_send.at[s],
                recv_sem=bf_recv.at[s],
                device_id=(partner,),
                device_id_type=pl.DeviceIdType.MESH,
            )
            exch.start()
            exch.wait()
            amax = jnp.maximum(amax, bf_dst[s, 0, 0])

        scale = amax / 448.0
        a = out_ref[:, :] / scale
        q = a.astype(jnp.float8_e4m3fn).astype(jnp.float32)
        out_ref[:, :] = q * scale

    return pl.pallas_call(
        body,
        out_shape=jax.ShapeDtypeStruct((m_total, n_per), jnp.float32),
        in_specs=[
            pl.BlockSpec(memory_space=pltpu.VMEM),
            pl.BlockSpec(memory_space=pltpu.VMEM),
        ],
        out_specs=pl.BlockSpec(memory_space=pltpu.VMEM),
        scratch_shapes=[
            pltpu.VMEM((2, m_per, k), jnp.float32),
            pltpu.VMEM((2, m_per, k), jnp.float32),
            pltpu.SemaphoreType.DMA((R_HOPS,)),
            pltpu.SemaphoreType.DMA((R_HOPS,)),
            pltpu.SemaphoreType.DMA((L_HOPS,)),
            pltpu.SemaphoreType.DMA((L_HOPS,)),
            pltpu.SemaphoreType.REGULAR,
            pltpu.SemaphoreType.REGULAR,
            pltpu.VMEM((BF_STAGES, 8, 128), jnp.float32),
            pltpu.VMEM((BF_STAGES, 8, 128), jnp.float32),
            pltpu.SemaphoreType.DMA((BF_STAGES,)),
            pltpu.SemaphoreType.DMA((BF_STAGES,)),
        ],
        compiler_params=pltpu.CompilerParams(collective_id=0),
    )(x, w_mat)


# device time: 426893 ns/iter; 1.8697x vs baseline; 1.8697x over previous
import jax
import jax.numpy as jnp
from jax import lax
from jax.experimental import pallas as pl
from jax.experimental.pallas import tpu as pltpu

P = 32
R_HOPS = P // 2
L_HOPS = P // 2 - 1
BF_STAGES = 5

CYC = [0, 8, 16, 24, 25, 17, 9, 1, 2, 10, 18, 26, 29, 21, 13, 5,
       6, 14, 22, 30, 31, 23, 15, 7, 4, 12, 20, 28, 27, 19, 11, 3]


def _cyc_lookup(idx, table):
    v = jnp.int32(table[0])
    for j in range(1, P):
        v = jnp.where(idx == j, jnp.int32(table[j]), v)
    return v


def kernel(x, w_mat):
    m_per, k = x.shape
    _, n_per = w_mat.shape
    m_total = P * m_per

    def body(x_ref, w_ref, out_ref, buf_r, buf_l,
             send_r, recv_r, send_l, recv_l,
             credit_r, credit_l,
             bf_src, bf_dst, bf_send, bf_recv):
        my = lax.axis_index("i")
        pos = jnp.int32(0)
        for j in range(1, P):
            pos = jnp.where(my == CYC[j], jnp.int32(j), pos)
        right = _cyc_lookup(lax.rem(pos + 1, P), CYC)
        left = _cyc_lookup(lax.rem(pos + P - 1, P), CYC)

        barrier = pltpu.get_barrier_semaphore()
        for nbr in (left, right):
            pl.semaphore_signal(barrier, inc=1, device_id=(nbr,),
                                device_id_type=pl.DeviceIdType.MESH)
        pl.semaphore_wait(barrier, 2)

        w = w_ref[:, :]

        def gemm_relu(chunk):
            y = jnp.dot(chunk, w, preferred_element_type=jnp.float32)
            return jnp.maximum(y, 0.0)

        y0 = gemm_relu(x_ref[:, :])
        out_ref[pl.ds(my * m_per, m_per), :] = y0
        amax = jnp.max(y0)

        for h in range(R_HOPS):
            if h >= 2:
                pl.semaphore_wait(credit_r, 1)
            if 2 <= h < L_HOPS:
                pl.semaphore_wait(credit_l, 1)

            src_r = x_ref if h == 0 else buf_r.at[(h - 1) % 2]
            rdma_r = pltpu.make_async_remote_copy(
                src_ref=src_r,
                dst_ref=buf_r.at[h % 2],
                send_sem=send_r.at[h],
                recv_sem=recv_r.at[h],
                device_id=(right,),
                device_id_type=pl.DeviceIdType.MESH,
            )
            rdma_r.start()
            if h < L_HOPS:
                src_l = x_ref if h == 0 else buf_l.at[(h - 1) % 2]
                rdma_l = pltpu.make_async_remote_copy(
                    src_ref=src_l,
                    dst_ref=buf_l.at[h % 2],
                    send_sem=send_l.at[h],
                    recv_sem=recv_l.at[h],
                    device_id=(left,),
                    device_id_type=pl.DeviceIdType.MESH,
                )
                rdma_l.start()

            rdma_r.wait()
            if 1 <= h <= R_HOPS - 2:
                pl.semaphore_signal(credit_r, inc=1, device_id=(left,),
                                    device_id_type=pl.DeviceIdType.MESH)
            origin_r = _cyc_lookup(lax.rem(pos - 1 - h + P, P), CYC)
            y_r = gemm_relu(buf_r[h % 2])
            out_ref[pl.ds(origin_r * m_per, m_per), :] = y_r
            amax = jnp.maximum(amax, jnp.max(y_r))

            if h < L_HOPS:
                rdma_l.wait()
                if 1 <= h <= L_HOPS - 2:
                    pl.semaphore_signal(credit_l, inc=1, device_id=(right,),
                                        device_id_type=pl.DeviceIdType.MESH)
                origin_l = _cyc_lookup(lax.rem(pos + 1 + h, P), CYC)
                y_l = gemm_relu(buf_l[h % 2])
                out_ref[pl.ds(origin_l * m_per, m_per), :] = y_l
                amax = jnp.maximum(amax, jnp.max(y_l))

        for s in range(BF_STAGES):
            partner = my ^ (1 << s)
            bf_src[s] = jnp.full((8, 128), amax, dtype=jnp.float32)
            exch = pltpu.make_async_remote_copy(
                src_ref=bf_src.at[s],
                dst_ref=bf_dst.at[s],
                send_sem=bf_send.at[s],
                recv_sem=bf_recv.at[s],
                device_id=(partner,),
                device_id_type=pl.DeviceIdType.MESH,
            )
            exch.start()
            exch.wait()
            amax = jnp.maximum(amax, bf_dst[s, 0, 0])

        scale = amax / 448.0
        a = out_ref[:, :] / scale
        q = a.astype(jnp.float8_e4m3fn).astype(jnp.float32)
        out_ref[:, :] = q * scale

    return pl.pallas_call(
        body,
        out_shape=jax.ShapeDtypeStruct((m_total, n_per), jnp.float32),
        in_specs=[
            pl.BlockSpec(memory_space=pltpu.VMEM),
            pl.BlockSpec(memory_space=pltpu.VMEM),
        ],
        out_specs=pl.BlockSpec(memory_space=pltpu.VMEM),
        scratch_shapes=[
            pltpu.VMEM((2, m_per, k), jnp.float32),
            pltpu.VMEM((2, m_per, k), jnp.float32),
            pltpu.SemaphoreType.DMA((R_HOPS,)),
            pltpu.SemaphoreType.DMA((R_HOPS,)),
            pltpu.SemaphoreType.DMA((L_HOPS,)),
            pltpu.SemaphoreType.DMA((L_HOPS,)),
            pltpu.SemaphoreType.REGULAR,
            pltpu.SemaphoreType.REGULAR,
            pltpu.VMEM((BF_STAGES, 8, 128), jnp.float32),
            pltpu.VMEM((BF_STAGES, 8, 128), jnp.float32),
            pltpu.SemaphoreType.DMA((BF_STAGES,)),
            pltpu.SemaphoreType.DMA((BF_STAGES,)),
        ],
        compiler_params=pltpu.CompilerParams(collective_id=0),
    )(x, w_mat)


# device time: 418201 ns/iter; 1.9086x vs baseline; 1.0208x over previous
import jax
import jax.numpy as jnp
from jax import lax
from jax.experimental import pallas as pl
from jax.experimental.pallas import tpu as pltpu

P = 32
R_HOPS = P // 2
L_HOPS = P // 2 - 1
BF_STAGES = 5

CYC = [0, 8, 16, 24, 25, 17, 9, 1, 2, 10, 18, 26, 29, 21, 13, 5,
       6, 14, 22, 30, 31, 23, 15, 7, 4, 12, 20, 28, 27, 19, 11, 3]


def _cyc_lookup(idx, table):
    v = jnp.int32(table[0])
    for j in range(1, P):
        v = jnp.where(idx == j, jnp.int32(table[j]), v)
    return v


def kernel(x, w_mat):
    m_per, k = x.shape
    _, n_per = w_mat.shape
    m_total = P * m_per

    def body(x_ref, w_ref, out_ref, buf_r, buf_l,
             send_r, recv_r, send_l, recv_l,
             credit_r, credit_l,
             bf_src, bf_dst, bf_send, bf_recv):
        my = lax.axis_index("i")
        pos = jnp.int32(0)
        for j in range(1, P):
            pos = jnp.where(my == CYC[j], jnp.int32(j), pos)
        right = _cyc_lookup(lax.rem(pos + 1, P), CYC)
        left = _cyc_lookup(lax.rem(pos + P - 1, P), CYC)

        barrier = pltpu.get_barrier_semaphore()
        for nbr in (left, right):
            pl.semaphore_signal(barrier, inc=1, device_id=(nbr,),
                                device_id_type=pl.DeviceIdType.MESH)
        pl.semaphore_wait(barrier, 2)

        w = w_ref[:, :]

        def gemm_relu(chunk):
            y = jnp.dot(chunk, w, preferred_element_type=jnp.float32)
            return jnp.maximum(y, 0.0)

        y0 = gemm_relu(x_ref[:, :])
        out_ref[pl.ds(my * m_per, m_per), :] = y0
        amax = jnp.max(y0)

        def mk(src, dst_slot, sems_s, sems_r, h, dev):
            return pltpu.make_async_remote_copy(
                src_ref=src, dst_ref=dst_slot,
                send_sem=sems_s.at[h], recv_sem=sems_r.at[h],
                device_id=(dev,), device_id_type=pl.DeviceIdType.MESH,
            )

        rdmas_r = {0: mk(x_ref, buf_r.at[0], send_r, recv_r, 0, right)}
        rdmas_l = {0: mk(x_ref, buf_l.at[0], send_l, recv_l, 0, left)}
        rdmas_r[0].start()
        rdmas_l[0].start()

        for h in range(R_HOPS):
            rdmas_r[h].wait_recv()
            rdmas_r[h].wait_send()
            if 1 <= h <= R_HOPS - 2:
                pl.semaphore_signal(credit_r, inc=1, device_id=(left,),
                                    device_id_type=pl.DeviceIdType.MESH)
            if h + 1 < R_HOPS:
                if h + 1 >= 2:
                    pl.semaphore_wait(credit_r, 1)
                rdmas_r[h + 1] = mk(buf_r.at[h % 2], buf_r.at[(h + 1) % 2],
                                    send_r, recv_r, h + 1, right)
                rdmas_r[h + 1].start()

            if h < L_HOPS:
                rdmas_l[h].wait_recv()
                rdmas_l[h].wait_send()
                if 1 <= h <= L_HOPS - 2:
                    pl.semaphore_signal(credit_l, inc=1, device_id=(right,),
                                        device_id_type=pl.DeviceIdType.MESH)
                if h + 1 < L_HOPS:
                    if h + 1 >= 2:
                        pl.semaphore_wait(credit_l, 1)
                    rdmas_l[h + 1] = mk(buf_l.at[h % 2], buf_l.at[(h + 1) % 2],
                                        send_l, recv_l, h + 1, left)
                    rdmas_l[h + 1].start()

            origin_r = _cyc_lookup(lax.rem(pos - 1 - h + P, P), CYC)
            y_r = gemm_relu(buf_r[h % 2])
            out_ref[pl.ds(origin_r * m_per, m_per), :] = y_r
            amax = jnp.maximum(amax, jnp.max(y_r))

            if h < L_HOPS:
                origin_l = _cyc_lookup(lax.rem(pos + 1 + h, P), CYC)
                y_l = gemm_relu(buf_l[h % 2])
                out_ref[pl.ds(origin_l * m_per, m_per), :] = y_l
                amax = jnp.maximum(amax, jnp.max(y_l))

        for s in range(BF_STAGES):
            partner = my ^ (1 << s)
            bf_src[s] = jnp.full((8, 128), amax, dtype=jnp.float32)
            exch = pltpu.make_async_remote_copy(
                src_ref=bf_src.at[s],
                dst_ref=bf_dst.at[s],
                send_sem=bf_send.at[s],
                recv_sem=bf_recv.at[s],
                device_id=(partner,),
                device_id_type=pl.DeviceIdType.MESH,
            )
            exch.start()
            exch.wait()
            amax = jnp.maximum(amax, bf_dst[s, 0, 0])

        scale = amax / 448.0
        a = out_ref[:, :] / scale
        q = a.astype(jnp.float8_e4m3fn).astype(jnp.float32)
        out_ref[:, :] = q * scale

    return pl.pallas_call(
        body,
        out_shape=jax.ShapeDtypeStruct((m_total, n_per), jnp.float32),
        in_specs=[
            pl.BlockSpec(memory_space=pltpu.VMEM),
            pl.BlockSpec(memory_space=pltpu.VMEM),
        ],
        out_specs=pl.BlockSpec(memory_space=pltpu.VMEM),
        scratch_shapes=[
            pltpu.VMEM((2, m_per, k), jnp.float32),
            pltpu.VMEM((2, m_per, k), jnp.float32),
            pltpu.SemaphoreType.DMA((R_HOPS,)),
            pltpu.SemaphoreType.DMA((R_HOPS,)),
            pltpu.SemaphoreType.DMA((L_HOPS,)),
            pltpu.SemaphoreType.DMA((L_HOPS,)),
            pltpu.SemaphoreType.REGULAR,
            pltpu.SemaphoreType.REGULAR,
            pltpu.VMEM((BF_STAGES, 8, 128), jnp.float32),
            pltpu.VMEM((BF_STAGES, 8, 128), jnp.float32),
            pltpu.SemaphoreType.DMA((BF_STAGES,)),
            pltpu.SemaphoreType.DMA((BF_STAGES,)),
        ],
        compiler_params=pltpu.CompilerParams(collective_id=0),
    )(x, w_mat)


# device time: 375417 ns/iter; 2.1261x vs baseline; 1.1140x over previous
import jax
import jax.numpy as jnp
from jax import lax
from jax.experimental import pallas as pl
from jax.experimental.pallas import tpu as pltpu

P = 32
HOPS = P // 2
S = 4

_R_SUBS = {h: list(range(S)) for h in range(HOPS)}
_R_SUBS[HOPS - 1] = [0, 1]
_L_SUBS = {h: list(range(S)) for h in range(HOPS)}
_L_SUBS[HOPS - 1] = [2, 3]

CYC = [0, 8, 16, 24, 25, 17, 9, 1, 2, 10, 18, 26, 29, 21, 13, 5,
       6, 14, 22, 30, 31, 23, 15, 7, 4, 12, 20, 28, 27, 19, 11, 3]


def _cyc_lookup(idx, table):
    v = jnp.int32(table[0])
    for j in range(1, P):
        v = jnp.where(idx == j, jnp.int32(table[j]), v)
    return v


def kernel(x, w_mat):
    m_per, k = x.shape
    _, n_per = w_mat.shape
    m_total = P * m_per
    sub_m = m_per // S

    def body(x_ref, w_ref, out_ref, buf_r, buf_l,
             send_r, recv_r, send_l, recv_l,
             credit_r, credit_l,
             bf_src, bf_dst, bf_send, bf_recv):
        my = lax.axis_index("i")
        pos = jnp.int32(0)
        for j in range(1, P):
            pos = jnp.where(my == CYC[j], jnp.int32(j), pos)
        right = _cyc_lookup(lax.rem(pos + 1, P), CYC)
        left = _cyc_lookup(lax.rem(pos + P - 1, P), CYC)

        barrier = pltpu.get_barrier_semaphore()
        for nbr in (left, right):
            pl.semaphore_signal(barrier, inc=1, device_id=(nbr,),
                                device_id_type=pl.DeviceIdType.MESH)
        pl.semaphore_wait(barrier, 2)

        w = w_ref[:, :]

        def gemm_relu(chunk):
            y = jnp.dot(chunk, w, preferred_element_type=jnp.float32)
            return jnp.maximum(y, 0.0)

        y0 = gemm_relu(x_ref[:, :])
        out_ref[pl.ds(my * m_per, m_per), :] = y0
        amax = jnp.max(y0)

        rdmas_r = {}
        rdmas_l = {}
        for j in range(S):
            sub = pl.ds(j * sub_m, sub_m)
            rdmas_r[(0, j)] = pltpu.make_async_remote_copy(
                src_ref=x_ref.at[sub], dst_ref=buf_r.at[0, sub],
                send_sem=send_r.at[0, j], recv_sem=recv_r.at[0, j],
                device_id=(right,), device_id_type=pl.DeviceIdType.MESH,
            )
            rdmas_r[(0, j)].start()
            rdmas_l[(0, j)] = pltpu.make_async_remote_copy(
                src_ref=x_ref.at[sub], dst_ref=buf_l.at[0, sub],
                send_sem=send_l.at[0, j], recv_sem=recv_l.at[0, j],
                device_id=(left,), device_id_type=pl.DeviceIdType.MESH,
            )
            rdmas_l[(0, j)].start()

        def hop_subs(h, subs_map, buf, sems_s, sems_r, rdmas, credit,
                     up_dev, down_dev):
            for j in subs_map[h]:
                sub = pl.ds(j * sub_m, sub_m)
                rdmas[(h, j)].wait_recv()
                rdmas[(h, j)].wait_send()
                if h >= 1 and h + 1 < HOPS and j in subs_map[h + 1]:
                    pl.semaphore_signal(credit, inc=1, device_id=(up_dev,),
                                        device_id_type=pl.DeviceIdType.MESH)
                if h + 1 < HOPS and j in subs_map[h + 1]:
                    if h + 1 >= 2:
                        pl.semaphore_wait(credit, 1)
                    nxt = pltpu.make_async_remote_copy(
                        src_ref=buf.at[h % 2, sub],
                        dst_ref=buf.at[(h + 1) % 2, sub],
                        send_sem=sems_s.at[(h + 1) % 2, j],
                        recv_sem=sems_r.at[(h + 1) % 2, j],
                        device_id=(down_dev,),
                        device_id_type=pl.DeviceIdType.MESH,
                    )
                    nxt.start()
                    rdmas[(h + 1, j)] = nxt

        half_m = m_per // 2
        for h in range(HOPS):
            hop_subs(h, _R_SUBS, buf_r, send_r, recv_r, rdmas_r, credit_r,
                     left, right)
            hop_subs(h, _L_SUBS, buf_l, send_l, recv_l, rdmas_l, credit_l,
                     right, left)

            origin_r = _cyc_lookup(lax.rem(pos - 1 - h + P, P), CYC)
            origin_l = _cyc_lookup(lax.rem(pos + 1 + h, P), CYC)
            if h < HOPS - 1:
                y_r = gemm_relu(buf_r[h % 2])
                out_ref[pl.ds(origin_r * m_per, m_per), :] = y_r
                amax = jnp.maximum(amax, jnp.max(y_r))
                y_l = gemm_relu(buf_l[h % 2])
                out_ref[pl.ds(origin_l * m_per, m_per), :] = y_l
                amax = jnp.maximum(amax, jnp.max(y_l))
            else:
                y_r = gemm_relu(buf_r[h % 2, : half_m])
                out_ref[pl.ds(origin_r * m_per, half_m), :] = y_r
                amax = jnp.maximum(amax, jnp.max(y_r))
                y_l = gemm_relu(buf_l[h % 2, half_m:])
                out_ref[pl.ds(origin_l * m_per + half_m, half_m), :] = y_l
                amax = jnp.maximum(amax, jnp.max(y_l))

        for s in range(5):
            partner = my ^ (1 << s)
            bf_src[s] = jnp.full((8, 128), amax, dtype=jnp.float32)
            exch = pltpu.make_async_remote_copy(
                src_ref=bf_src.at[s],
                dst_ref=bf_dst.at[s],
                send_sem=bf_send.at[s],
                recv_sem=bf_recv.at[s],
                device_id=(partner,),
                device_id_type=pl.DeviceIdType.MESH,
            )
            exch.start()
            exch.wait()
            amax = jnp.maximum(amax, bf_dst[s, 0, 0])

        scale = amax / 448.0
        a = out_ref[:, :] / scale
        q = a.astype(jnp.float8_e4m3fn).astype(jnp.float32)
        out_ref[:, :] = q * scale

    return pl.pallas_call(
        body,
        out_shape=jax.ShapeDtypeStruct((m_total, n_per), jnp.float32),
        in_specs=[
            pl.BlockSpec(memory_space=pltpu.VMEM),
            pl.BlockSpec(memory_space=pltpu.VMEM),
        ],
        out_specs=pl.BlockSpec(memory_space=pltpu.VMEM),
        scratch_shapes=[
            pltpu.VMEM((2, m_per, k), jnp.float32),
            pltpu.VMEM((2, m_per, k), jnp.float32),
            pltpu.SemaphoreType.DMA((2, S)),
            pltpu.SemaphoreType.DMA((2, S)),
            pltpu.SemaphoreType.DMA((2, S)),
            pltpu.SemaphoreType.DMA((2, S)),
            pltpu.SemaphoreType.REGULAR,
            pltpu.SemaphoreType.REGULAR,
            pltpu.VMEM((5, 8, 128), jnp.float32),
            pltpu.VMEM((5, 8, 128), jnp.float32),
            pltpu.SemaphoreType.DMA((5,)),
            pltpu.SemaphoreType.DMA((5,)),
        ],
        compiler_params=pltpu.CompilerParams(collective_id=0),
    )(x, w_mat)
